# Initial kernel scaffold; baseline (speedup 1.0000x reference)
import jax
import jax.numpy as jnp
from jax.experimental import pallas as pl


def kernel(targets, strides, grids, outputs, regs, masks, use_augs):
    raise NotImplementedError("write your pallas kernel here")


# speedup vs baseline: 8.1044x; 8.1044x over previous
import jax
import jax.numpy as jnp
from jax.experimental import pallas as pl
from jax.experimental.pallas import tpu as pltpu

_A = 8400
_AP = 8448
_G = 50
_GP = 64
_B = 16


def _body(anc_ref, tgt_ref, out_ref, msk_ref, acc_ref, per_ref):
    i = pl.program_id(0)
    f32 = jnp.float32
    INF = f32(jnp.inf)
    BIG = f32(1e9)

    anc = anc_ref[...]
    ot = out_ref[0]
    tg = tgt_ref[0]
    mk = msk_ref[0]

    aidx = jax.lax.broadcasted_iota(jnp.int32, (1, _AP), 1)
    aidxf = aidx.astype(f32)
    avalid = aidx < _A
    avf = avalid.astype(f32)
    gidxf = jax.lax.broadcasted_iota(jnp.int32, (_GP, 1), 0).astype(f32)

    gx = anc[0:1, :]
    gy = anc[1:2, :]
    st = anc[2:3, :]
    xc = (gx + 0.5) * st
    yc = (gy + 0.5) * st

    tcx = tg[:, 1:2]
    tcy = tg[:, 2:3]
    tw = tg[:, 3:4]
    th = tg[:, 4:5]
    valid = tg[:, 0:1] > 0.0

    in_box = ((xc > tcx - tw * 0.5) & (xc < tcx + tw * 0.5)
              & (yc > tcy - th * 0.5) & (yc < tcy + th * 0.5)) & valid & avalid
    r = 2.5
    in_ctr = ((xc > tcx - r * st) & (xc < tcx + r * st)
              & (yc > tcy - r * st) & (yc < tcy + r * st)) & valid & avalid
    cand = jnp.any(in_box | in_ctr, axis=0, keepdims=True)
    both = in_box & in_ctr

    px = ot[0:1]
    py = ot[1:2]
    pw = ot[2:3]
    ph = ot[3:4]
    tlx = jnp.maximum(tcx - tw * 0.5, px - pw * 0.5)
    tly = jnp.maximum(tcy - th * 0.5, py - ph * 0.5)
    brx = jnp.minimum(tcx + tw * 0.5, px + pw * 0.5)
    bry = jnp.minimum(tcy + th * 0.5, py + ph * 0.5)
    inter = jnp.maximum(brx - tlx, 0.0) * jnp.maximum(bry - tly, 0.0)
    iou_full = inter / (tw * th + pw * ph - inter + 1e-8)
    ious = jnp.where(cand, iou_full, 0.0)

    sig5 = 1.0 / (1.0 + jnp.exp(-ot[5:6]))
    sig6 = 1.0 / (1.0 + jnp.exp(-ot[6:7]))
    p = jnp.sqrt(sig6 * sig5 + 1e-12)
    cls_cost = -jnp.log(p + 1e-8)
    cost = cls_cost + 3.0 * (-jnp.log(ious + 1e-8)) \
        + 100000.0 * jnp.where(both, 0.0, 1.0)
    cost = jnp.where(cand, cost, INF)

    act = jnp.full((_GP, _AP), True)
    s = jnp.zeros((_GP, 1), f32)
    for _ in range(10):
        m = jnp.max(jnp.where(act, ious, -1.0), axis=1, keepdims=True)
        hit = act & (ious == m)
        fidx = jnp.min(jnp.where(hit, aidxf, BIG), axis=1, keepdims=True)
        act = act & (aidxf != fidx)
        s = s + m
    dksm1 = jnp.clip(s.astype(jnp.int32), 1, None) - 1

    act2 = jnp.full((_GP, _AP), True)
    thr_v = jnp.zeros((_GP, 1), f32)
    thr_i = jnp.zeros((_GP, 1), f32)
    for j in range(10):
        m = jnp.min(jnp.where(act2, cost, INF), axis=1, keepdims=True)
        hit = act2 & (cost == m)
        fidx = jnp.min(jnp.where(hit, aidxf, BIG), axis=1, keepdims=True)
        act2 = act2 & (aidxf != fidx)
        sel = dksm1 == j
        thr_v = jnp.where(sel, m, thr_v)
        thr_i = jnp.where(sel, fidx, thr_i)

    matching = (cand & valid & (
        (cost < thr_v) | ((cost == thr_v) & (aidxf <= thr_i)))).astype(f32)

    multi = jnp.sum(matching, axis=0, keepdims=True) > 1.0
    cost_v = jnp.where(valid, cost, INF)
    minc = jnp.min(cost_v, axis=0, keepdims=True)
    best = jnp.min(jnp.where(cost_v == minc, gidxf, BIG), axis=0, keepdims=True)
    mf = jnp.where(multi, (gidxf == best).astype(f32), matching)
    fgf = (jnp.sum(mf, axis=0, keepdims=True) > 0.0).astype(f32)
    pred_ious = jnp.sum(mf * ious, axis=0, keepdims=True)
    num_fg = jnp.sum(fgf)

    mbx = jnp.sum(mf * tcx, axis=0, keepdims=True)
    mby = jnp.sum(mf * tcy, axis=0, keepdims=True)
    mbw = jnp.sum(mf * tw, axis=0, keepdims=True)
    mbh = jnp.sum(mf * th, axis=0, keepdims=True)

    tlx2 = jnp.maximum(px - pw * 0.5, mbx - mbw * 0.5)
    tly2 = jnp.maximum(py - ph * 0.5, mby - mbh * 0.5)
    brx2 = jnp.minimum(px + pw * 0.5, mbx + mbw * 0.5)
    bry2 = jnp.minimum(py + ph * 0.5, mby + mbh * 0.5)
    inter2 = jnp.maximum(brx2 - tlx2, 0.0) * jnp.maximum(bry2 - tly2, 0.0)
    iou2 = inter2 / (pw * ph + mbw * mbh - inter2 + 1e-16)
    l_iou = 5.0 * jnp.sum((1.0 - iou2 * iou2) * fgf)

    def bce(x, t):
        return jnp.maximum(x, 0.0) - x * t + jnp.log(1.0 + jnp.exp(-jnp.abs(x)))

    l_obj = jnp.sum(bce(ot[5:6], fgf) * avf)
    l_cls = jnp.sum(bce(ot[6:7], pred_ious) * fgf)
    imf = ((mk[0:1] + mk[1:2]) > 0.0).astype(f32) * avf
    l_mask = jnp.sum(bce(ot[4:5], mk[1:2]) * imf)
    num_m = jnp.sum(imf)
    num_gt = jnp.sum(valid.astype(f32))
    has_gt = num_gt > 0.0

    li = jax.lax.broadcasted_iota(jnp.int32, (1, 128), 1)

    @pl.when(i == 0)
    def _():
        acc_ref[...] = jnp.zeros((1, 128), f32)

    def sc(k, v):
        return jnp.where(li == k, v, 0.0)

    zero = f32(0.0)
    contrib = (sc(0, jnp.where(has_gt, l_iou, zero))
               + sc(1, jnp.where(has_gt, l_obj, zero))
               + sc(2, jnp.where(has_gt, l_cls, zero))
               + sc(3, jnp.where(has_gt, l_mask, zero))
               + sc(4, jnp.where(has_gt, num_fg, zero))
               + sc(5, jnp.where(has_gt, num_gt, zero))
               + sc(6, jnp.where(has_gt, num_m, zero)))
    acc_ref[...] = acc_ref[...] + contrib

    per_ref[0] = sc(0, jnp.where(
        has_gt, (l_obj + l_cls) / jnp.maximum(num_fg, 1.0), f32(-1.0)))

    @pl.when(i == _B - 1)
    def _():
        a = acc_ref[...]

        def lane(k):
            return jnp.sum(jnp.where(li == k, a, 0.0))

        nf = jnp.maximum(lane(4), 1.0)
        ng = jnp.maximum(lane(5), 1.0)
        nm = jnp.maximum(lane(6), 1.0)
        iou_l = lane(0) / nf
        obj_l = lane(1) / nf
        cls_l = lane(2) / nf
        mask_l = lane(3) / nm * 2.0
        total = iou_l + obj_l + cls_l + mask_l
        acc_ref[...] = (sc(0, total) + sc(1, iou_l) + sc(2, obj_l)
                        + sc(4, cls_l) + sc(5, mask_l) + sc(6, nf / ng))


def kernel(targets, strides, grids, outputs, regs, masks, use_augs):
    f32 = jnp.float32
    B = outputs.shape[0]

    anc = jnp.zeros((8, _AP), f32)
    anc = anc.at[0, :_A].set(grids[:, 0].astype(f32))
    anc = anc.at[1, :_A].set(grids[:, 1].astype(f32))
    anc = anc.at[2, :_A].set(strides.astype(f32))

    tgt = jnp.zeros((B, _GP, 8), f32).at[:, :_G, :5].set(targets)
    outs = jnp.zeros((B, 8, _AP), f32).at[:, :7, :_A].set(
        outputs.transpose(0, 2, 1))

    ms = []
    for s, step in [(80, 2), (40, 4), (20, 8)]:
        ms.append(masks[:, ::step, ::step, :].reshape(B, s * s, 2))
    mr = jnp.concatenate(ms, 1)
    msk = jnp.zeros((B, 8, _AP), f32).at[:, :2, :_A].set(mr.transpose(0, 2, 1))

    acc, per = pl.pallas_call(
        _body,
        grid=(B,),
        in_specs=[
            pl.BlockSpec((8, _AP), lambda i: (0, 0)),
            pl.BlockSpec((1, _GP, 8), lambda i: (i, 0, 0)),
            pl.BlockSpec((1, 8, _AP), lambda i: (i, 0, 0)),
            pl.BlockSpec((1, 8, _AP), lambda i: (i, 0, 0)),
        ],
        out_specs=[
            pl.BlockSpec((1, 128), lambda i: (0, 0)),
            pl.BlockSpec((1, 1, 128), lambda i: (i, 0, 0)),
        ],
        out_shape=[
            jax.ShapeDtypeStruct((1, 128), f32),
            jax.ShapeDtypeStruct((B, 1, 128), f32),
        ],
        compiler_params=pltpu.CompilerParams(
            dimension_semantics=("arbitrary",)),
    )(anc, tgt, outs, msk)

    return acc[0, :7], per[:, 0, 0]


# speedup vs baseline: 15.7910x; 1.9485x over previous
import functools

import jax
import jax.numpy as jnp
from jax import lax
from jax.experimental import pallas as pl
from jax.experimental.pallas import tpu as pltpu
from jax.experimental.pallas import tpu_sc as plsc

_A = 8400
_AP = 8448
_G = 50
_GP = 64
_B = 16
_R = _B * _GP
_NW = 32
_RW = _R // _NW
_NCH = _AP // 16


def _body_a(anc_ref, tgt_ref, out_ref, cost_ref, iou_ref):
    f32 = jnp.float32
    INF = f32(jnp.inf)

    anc = anc_ref[...]
    ot = out_ref[0]
    tg = tgt_ref[0]

    aidx = jax.lax.broadcasted_iota(jnp.int32, (1, _AP), 1)
    avalid = aidx < _A

    gx = anc[0:1, :]
    gy = anc[1:2, :]
    st = anc[2:3, :]
    xc = (gx + 0.5) * st
    yc = (gy + 0.5) * st

    tcx = tg[:, 1:2]
    tcy = tg[:, 2:3]
    tw = tg[:, 3:4]
    th = tg[:, 4:5]
    valid = tg[:, 0:1] > 0.0

    in_box = ((xc > tcx - tw * 0.5) & (xc < tcx + tw * 0.5)
              & (yc > tcy - th * 0.5) & (yc < tcy + th * 0.5)) & valid & avalid
    r = 2.5
    in_ctr = ((xc > tcx - r * st) & (xc < tcx + r * st)
              & (yc > tcy - r * st) & (yc < tcy + r * st)) & valid & avalid
    cand = jnp.any(in_box | in_ctr, axis=0, keepdims=True)
    both = in_box & in_ctr

    px = ot[0:1]
    py = ot[1:2]
    pw = ot[2:3]
    ph = ot[3:4]
    tlx = jnp.maximum(tcx - tw * 0.5, px - pw * 0.5)
    tly = jnp.maximum(tcy - th * 0.5, py - ph * 0.5)
    brx = jnp.minimum(tcx + tw * 0.5, px + pw * 0.5)
    bry = jnp.minimum(tcy + th * 0.5, py + ph * 0.5)
    inter = jnp.maximum(brx - tlx, 0.0) * jnp.maximum(bry - tly, 0.0)
    iou_full = inter / (tw * th + pw * ph - inter + 1e-8)
    ious = jnp.where(cand, iou_full, 0.0)

    sig5 = 1.0 / (1.0 + jnp.exp(-ot[5:6]))
    sig6 = 1.0 / (1.0 + jnp.exp(-ot[6:7]))
    p = jnp.sqrt(sig6 * sig5 + 1e-12)
    cls_cost = -jnp.log(p + 1e-8)
    cost = cls_cost + 3.0 * (-jnp.log(ious + 1e-8)) \
        + 100000.0 * jnp.where(both, 0.0, 1.0)
    cost_ref[0] = jnp.where(cand, cost, INF)
    iou_ref[0] = ious


def _sc_mesh():
    return plsc.VectorSubcoreMesh(core_axis_name="c", subcore_axis_name="s")


def _count(mask):
    return plsc.cumsum(jnp.where(mask, 1, 0))[15]


def _sc_thr_body(cost_hbm, iou_hbm, out_hbm, crow, irow, orow):
    f32 = jnp.float32
    INF = f32(jnp.inf)
    wid = lax.axis_index("s") * 2 + lax.axis_index("c")
    iota = lax.broadcasted_iota(jnp.int32, (16,), 0)

    def row_body(rl, _):
        row = wid * _RW + rl
        pltpu.sync_copy(cost_hbm.at[row], crow)
        pltpu.sync_copy(iou_hbm.at[row], irow)

        def pass_a(j, carry):
            ct, it = carry
            ch = crow[pl.ds(j * 16, 16)]
            ih = irow[pl.ds(j * 16, 16)]

            def merge_c(t):
                chs, _ = plsc.sort_key_val(ch, ch, descending=True)
                m = jnp.minimum(t, chs)
                t2, _ = plsc.sort_key_val(m, m)
                return t2

            def merge_i(t):
                ihs, _ = plsc.sort_key_val(ih, ih)
                m = jnp.maximum(t, ihs)
                t2, _ = plsc.sort_key_val(m, m, descending=True)
                return t2

            ct = lax.cond(_count(ch < ct[15]) > 0, merge_c, lambda t: t, ct)
            it = lax.cond(_count(ih > it[15]) > 0, merge_i, lambda t: t, it)
            return ct, it

        ct, it = lax.fori_loop(
            0, _NCH, pass_a,
            (jnp.full((16,), INF), jnp.full((16,), -INF)))

        s10 = plsc.cumsum(it)[9]
        s10v = jnp.zeros((16,), f32) + s10
        flo = _count(s10v >= (iota + 1).astype(f32))
        dks = jnp.clip(flo, 1, 10)
        thr_v = ct.at[jnp.zeros((16,), jnp.int32) + (dks - 1)].get(
            mode="promise_in_bounds")[0]

        def pass_b1(j, acc):
            ch = crow[pl.ds(j * 16, 16)]
            return acc + _count(ch < thr_v)

        c_less = lax.fori_loop(0, _NCH, pass_b1, jnp.int32(0))
        k_needed = dks - c_less

        def pass_b2(j, carry):
            cum, found = carry
            ch = crow[pl.ds(j * 16, 16)]
            eq = ch == thr_v
            cs = plsc.cumsum(jnp.where(eq, 1, 0))
            cnt = cs[15]
            tgt = eq & (cs == (k_needed - cum))
            lane = plsc.cumsum(jnp.where(tgt, iota, 0))[15]
            hit = (cum < k_needed) & (cum + cnt >= k_needed)
            found = jnp.where(hit, j * 16 + lane, found)
            return cum + cnt, found

        _, thr_i = lax.fori_loop(
            0, _NCH, pass_b2, (jnp.int32(0), jnp.int32(0)))

        out_vec = (jnp.where(iota == 0, thr_v, 0.0)
                   + jnp.where(iota == 1, thr_i.astype(f32), 0.0)
                   + jnp.where(iota == 2, dks.astype(f32), 0.0))
        orow[...] = out_vec
        pltpu.sync_copy(orow, out_hbm.at[row])
        return 0

    lax.fori_loop(0, _RW, row_body, 0)


def _body_c(anc_ref, tgt_ref, out_ref, msk_ref, cost_ref, iou_ref, thr_ref,
            acc_ref, per_ref):
    i = pl.program_id(0)
    f32 = jnp.float32
    INF = f32(jnp.inf)
    BIG = f32(1e9)

    ot = out_ref[0]
    tg = tgt_ref[0]
    mk = msk_ref[0]
    cost = cost_ref[0]
    ious = iou_ref[0]
    tr = thr_ref[0]
    thr_v = tr[:, 0:1]
    thr_i = tr[:, 1:2]

    aidx = jax.lax.broadcasted_iota(jnp.int32, (1, _AP), 1)
    aidxf = aidx.astype(f32)
    avalid = aidx < _A
    avf = avalid.astype(f32)
    gidxf = jax.lax.broadcasted_iota(jnp.int32, (_GP, 1), 0).astype(f32)

    tcx = tg[:, 1:2]
    tcy = tg[:, 2:3]
    tw = tg[:, 3:4]
    th = tg[:, 4:5]
    valid = tg[:, 0:1] > 0.0

    matching = (((cost < thr_v) | ((cost == thr_v) & (aidxf <= thr_i)))
                & (cost < INF) & valid).astype(f32)

    multi = jnp.sum(matching, axis=0, keepdims=True) > 1.0
    cost_v = jnp.where(valid, cost, INF)
    minc = jnp.min(cost_v, axis=0, keepdims=True)
    best = jnp.min(jnp.where(cost_v == minc, gidxf, BIG), axis=0, keepdims=True)
    mf = jnp.where(multi, (gidxf == best).astype(f32), matching)

    fgf = (jnp.sum(mf, axis=0, keepdims=True) > 0.0).astype(f32)
    pred_ious = jnp.sum(mf * ious, axis=0, keepdims=True)
    num_fg = jnp.sum(fgf)

    mbx = jnp.sum(mf * tcx, axis=0, keepdims=True)
    mby = jnp.sum(mf * tcy, axis=0, keepdims=True)
    mbw = jnp.sum(mf * tw, axis=0, keepdims=True)
    mbh = jnp.sum(mf * th, axis=0, keepdims=True)

    px = ot[0:1]
    py = ot[1:2]
    pw = ot[2:3]
    ph = ot[3:4]
    tlx2 = jnp.maximum(px - pw * 0.5, mbx - mbw * 0.5)
    tly2 = jnp.maximum(py - ph * 0.5, mby - mbh * 0.5)
    brx2 = jnp.minimum(px + pw * 0.5, mbx + mbw * 0.5)
    bry2 = jnp.minimum(py + ph * 0.5, mby + mbh * 0.5)
    inter2 = jnp.maximum(brx2 - tlx2, 0.0) * jnp.maximum(bry2 - tly2, 0.0)
    iou2 = inter2 / (pw * ph + mbw * mbh - inter2 + 1e-16)
    l_iou = 5.0 * jnp.sum((1.0 - iou2 * iou2) * fgf)

    def bce(x, t):
        return jnp.maximum(x, 0.0) - x * t + jnp.log(1.0 + jnp.exp(-jnp.abs(x)))

    l_obj = jnp.sum(bce(ot[5:6], fgf) * avf)
    l_cls = jnp.sum(bce(ot[6:7], pred_ious) * fgf)
    imf = ((mk[0:1] + mk[1:2]) > 0.0).astype(f32) * avf
    l_mask = jnp.sum(bce(ot[4:5], mk[1:2]) * imf)
    num_m = jnp.sum(imf)
    num_gt = jnp.sum(valid.astype(f32))
    has_gt = num_gt > 0.0

    li = jax.lax.broadcasted_iota(jnp.int32, (1, 128), 1)

    @pl.when(i == 0)
    def _():
        acc_ref[...] = jnp.zeros((1, 128), f32)

    def sc(k, v):
        return jnp.where(li == k, v, 0.0)

    zero = f32(0.0)
    contrib = (sc(0, jnp.where(has_gt, l_iou, zero))
               + sc(1, jnp.where(has_gt, l_obj, zero))
               + sc(2, jnp.where(has_gt, l_cls, zero))
               + sc(3, jnp.where(has_gt, l_mask, zero))
               + sc(4, jnp.where(has_gt, num_fg, zero))
               + sc(5, jnp.where(has_gt, num_gt, zero))
               + sc(6, jnp.where(has_gt, num_m, zero)))
    acc_ref[...] = acc_ref[...] + contrib

    per_ref[0] = sc(0, jnp.where(
        has_gt, (l_obj + l_cls) / jnp.maximum(num_fg, 1.0), f32(-1.0)))

    @pl.when(i == _B - 1)
    def _():
        a = acc_ref[...]

        def lane(k):
            return jnp.sum(jnp.where(li == k, a, 0.0))

        nf = jnp.maximum(lane(4), 1.0)
        ng = jnp.maximum(lane(5), 1.0)
        nm = jnp.maximum(lane(6), 1.0)
        iou_l = lane(0) / nf
        obj_l = lane(1) / nf
        cls_l = lane(2) / nf
        mask_l = lane(3) / nm * 2.0
        total = iou_l + obj_l + cls_l + mask_l
        acc_ref[...] = (sc(0, total) + sc(1, iou_l) + sc(2, obj_l)
                        + sc(4, cls_l) + sc(5, mask_l) + sc(6, nf / ng))


def kernel(targets, strides, grids, outputs, regs, masks, use_augs):
    f32 = jnp.float32
    B = outputs.shape[0]

    anc = jnp.zeros((8, _AP), f32)
    anc = anc.at[0, :_A].set(grids[:, 0].astype(f32))
    anc = anc.at[1, :_A].set(grids[:, 1].astype(f32))
    anc = anc.at[2, :_A].set(strides.astype(f32))

    tgt = jnp.zeros((B, _GP, 8), f32).at[:, :_G, :5].set(targets)
    outs = jnp.zeros((B, 8, _AP), f32).at[:, :7, :_A].set(
        outputs.transpose(0, 2, 1))

    ms = []
    for s, step in [(80, 2), (40, 4), (20, 8)]:
        ms.append(masks[:, ::step, ::step, :].reshape(B, s * s, 2))
    mr = jnp.concatenate(ms, 1)
    msk = jnp.zeros((B, 8, _AP), f32).at[:, :2, :_A].set(mr.transpose(0, 2, 1))

    cost, iou = pl.pallas_call(
        _body_a,
        grid=(B,),
        in_specs=[
            pl.BlockSpec((8, _AP), lambda i: (0, 0)),
            pl.BlockSpec((1, _GP, 8), lambda i: (i, 0, 0)),
            pl.BlockSpec((1, 8, _AP), lambda i: (i, 0, 0)),
        ],
        out_specs=[
            pl.BlockSpec((1, _GP, _AP), lambda i: (i, 0, 0)),
            pl.BlockSpec((1, _GP, _AP), lambda i: (i, 0, 0)),
        ],
        out_shape=[
            jax.ShapeDtypeStruct((B, _GP, _AP), f32),
            jax.ShapeDtypeStruct((B, _GP, _AP), f32),
        ],
        compiler_params=pltpu.CompilerParams(
            dimension_semantics=("arbitrary",)),
    )(anc, tgt, outs)

    sc_thr = functools.partial(
        pl.kernel,
        mesh=_sc_mesh(),
        compiler_params=pltpu.CompilerParams(needs_layout_passes=False),
        out_type=jax.ShapeDtypeStruct((_R, 16), f32),
        scratch_types=[
            pltpu.VMEM((_AP,), f32),
            pltpu.VMEM((_AP,), f32),
            pltpu.VMEM((16,), f32),
        ],
    )(_sc_thr_body)
    thr = sc_thr(cost.reshape(_R, _AP), iou.reshape(_R, _AP))

    acc, per = pl.pallas_call(
        _body_c,
        grid=(B,),
        in_specs=[
            pl.BlockSpec((8, _AP), lambda i: (0, 0)),
            pl.BlockSpec((1, _GP, 8), lambda i: (i, 0, 0)),
            pl.BlockSpec((1, 8, _AP), lambda i: (i, 0, 0)),
            pl.BlockSpec((1, 8, _AP), lambda i: (i, 0, 0)),
            pl.BlockSpec((1, _GP, _AP), lambda i: (i, 0, 0)),
            pl.BlockSpec((1, _GP, _AP), lambda i: (i, 0, 0)),
            pl.BlockSpec((1, _GP, 16), lambda i: (i, 0, 0)),
        ],
        out_specs=[
            pl.BlockSpec((1, 128), lambda i: (0, 0)),
            pl.BlockSpec((1, 1, 128), lambda i: (i, 0, 0)),
        ],
        out_shape=[
            jax.ShapeDtypeStruct((1, 128), f32),
            jax.ShapeDtypeStruct((B, 1, 128), f32),
        ],
        compiler_params=pltpu.CompilerParams(
            dimension_semantics=("arbitrary",)),
    )(anc, tgt, outs, msk, cost, iou, thr.reshape(B, _GP, 16))

    return acc[0, :7], per[:, 0, 0]
